# baseline (speedup 1.0000x reference)
import functools

import jax
import jax.numpy as jnp
from jax.experimental import pallas as pl

K = 8


def _router_kernel(x_ref, w_ref, b_ref, logits_ref, idx_ref, wts_ref):
    T = x_ref.shape[0]
    E = w_ref.shape[0]
    logits = jax.lax.dot_general(
        x_ref[...], w_ref[...],
        dimension_numbers=(((1,), (1,)), ((), ())),
        preferred_element_type=jnp.float32,
    )
    logits_ref[...] = logits
    s = jax.nn.sigmoid(logits)
    sel = s + b_ref[...]
    iota = jax.lax.broadcasted_iota(jnp.int32, (T, E), 1)
    work = sel
    cols_i = []
    cols_w = []
    for _ in range(K):
        m = jnp.max(work, axis=1, keepdims=True)
        am = jnp.min(jnp.where(work == m, iota, E), axis=1, keepdims=True)
        onehot = iota == am
        cols_i.append(am)
        cols_w.append(jnp.sum(jnp.where(onehot, s, 0.0), axis=1, keepdims=True))
        work = jnp.where(onehot, -jnp.inf, work)
    idx = jnp.concatenate(cols_i, axis=1)
    wts = jnp.concatenate(cols_w, axis=1)
    wts = wts / (jnp.sum(wts, axis=1, keepdims=True) + 1e-20)
    idx_ref[...] = idx
    wts_ref[...] = wts


@functools.partial(jax.jit, static_argnames=())
def kernel(x, weight, balance_bias):
    orig_shape = x.shape
    D = orig_shape[-1]
    x_flat = x.reshape(-1, D).astype(jnp.float32)
    N = x_flat.shape[0]
    E = weight.shape[0]
    T = 256
    if N % T != 0:
        T = N
    bias2d = balance_bias.astype(jnp.float32).reshape(1, E)
    grid = (N // T,)
    logits, idx, wts = pl.pallas_call(
        _router_kernel,
        grid=grid,
        in_specs=[
            pl.BlockSpec((T, D), lambda i: (i, 0)),
            pl.BlockSpec((E, D), lambda i: (0, 0)),
            pl.BlockSpec((1, E), lambda i: (0, 0)),
        ],
        out_specs=[
            pl.BlockSpec((T, E), lambda i: (i, 0)),
            pl.BlockSpec((T, K), lambda i: (i, 0)),
            pl.BlockSpec((T, K), lambda i: (i, 0)),
        ],
        out_shape=[
            jax.ShapeDtypeStruct((N, E), jnp.float32),
            jax.ShapeDtypeStruct((N, K), jnp.int32),
            jax.ShapeDtypeStruct((N, K), jnp.float32),
        ],
    )(x_flat, weight.astype(jnp.float32), bias2d)
    return (idx, wts, logits)


# speedup vs baseline: 1.7723x; 1.7723x over previous
import functools

import jax
import jax.numpy as jnp
from jax.experimental import pallas as pl

K = 8


def _router_kernel(x_ref, w_ref, b_ref, logits_ref, idx_ref, wts_ref):
    T = x_ref.shape[0]
    E = w_ref.shape[0]
    dn = (((1,), (1,)), ((), ()))
    logits_t = jax.lax.dot_general(
        w_ref[...], x_ref[...], dn, preferred_element_type=jnp.float32)
    logits_ref[...] = logits_t
    s = jax.nn.sigmoid(logits_t)
    sel = s + b_ref[...]
    iota = jax.lax.broadcasted_iota(jnp.int32, (E, T), 0)
    work = sel
    rows_i = []
    rows_w = []
    for _ in range(K):
        m = jnp.max(work, axis=0, keepdims=True)
        am = jnp.min(jnp.where(work == m, iota, E), axis=0, keepdims=True)
        onehot = iota == am
        rows_i.append(am)
        rows_w.append(jnp.sum(jnp.where(onehot, s, 0.0), axis=0, keepdims=True))
        work = jnp.where(onehot, -jnp.inf, work)
    idx = jnp.concatenate(rows_i, axis=0)
    wts = jnp.concatenate(rows_w, axis=0)
    wts = wts / (jnp.sum(wts, axis=0, keepdims=True) + 1e-20)
    idx_ref[...] = idx
    wts_ref[...] = wts


@functools.partial(jax.jit, static_argnames=())
def kernel(x, weight, balance_bias):
    orig_shape = x.shape
    D = orig_shape[-1]
    x_flat = x.reshape(-1, D).astype(jnp.float32)
    N = x_flat.shape[0]
    E = weight.shape[0]
    T = 256
    if N % T != 0:
        T = N
    bias_col = balance_bias.astype(jnp.float32).reshape(E, 1)
    grid = (N // T,)
    logits_t, idx_t, wts_t = pl.pallas_call(
        _router_kernel,
        grid=grid,
        in_specs=[
            pl.BlockSpec((T, D), lambda i: (i, 0)),
            pl.BlockSpec((E, D), lambda i: (0, 0)),
            pl.BlockSpec((E, 1), lambda i: (0, 0)),
        ],
        out_specs=[
            pl.BlockSpec((E, T), lambda i: (0, i)),
            pl.BlockSpec((K, T), lambda i: (0, i)),
            pl.BlockSpec((K, T), lambda i: (0, i)),
        ],
        out_shape=[
            jax.ShapeDtypeStruct((E, N), jnp.float32),
            jax.ShapeDtypeStruct((K, N), jnp.int32),
            jax.ShapeDtypeStruct((K, N), jnp.float32),
        ],
    )(x_flat, weight.astype(jnp.float32), bias_col)
    return (idx_t.T, wts_t.T, logits_t.T)


# speedup vs baseline: 2.1667x; 1.2225x over previous
import functools

import jax
import jax.numpy as jnp
from jax.experimental import pallas as pl

K = 8


def _router_kernel(x_ref, w_ref, b_ref, logits_ref, idx_ref, wts_ref):
    T = x_ref.shape[0]
    E = w_ref.shape[0]
    dn = (((1,), (1,)), ((), ()))
    logits_t = jax.lax.dot_general(
        w_ref[...], x_ref[...], dn, preferred_element_type=jnp.float32)
    logits_ref[...] = logits_t
    s = jax.nn.sigmoid(logits_t)
    sel = s + b_ref[...]
    iota = jax.lax.broadcasted_iota(jnp.int32, (E, T), 0)
    work = sel
    rows_i = []
    rows_w = []
    for _ in range(K):
        m = jnp.max(work, axis=0, keepdims=True)
        am = jnp.min(jnp.where(work == m, iota, E), axis=0, keepdims=True)
        onehot = iota == am
        rows_i.append(am)
        rows_w.append(jnp.sum(jnp.where(onehot, s, 0.0), axis=0, keepdims=True))
        work = jnp.where(onehot, -jnp.inf, work)
    idx = jnp.concatenate(rows_i, axis=0)
    wts = jnp.concatenate(rows_w, axis=0)
    wts = wts / (jnp.sum(wts, axis=0, keepdims=True) + 1e-20)
    idx_ref[...] = idx
    wts_ref[...] = wts


@functools.partial(jax.jit, static_argnames=())
def kernel(x, weight, balance_bias):
    orig_shape = x.shape
    D = orig_shape[-1]
    x_flat = x.reshape(-1, D).astype(jnp.float32)
    N = x_flat.shape[0]
    E = weight.shape[0]
    T = 512
    if N % T != 0:
        T = N
    bias_col = balance_bias.astype(jnp.float32).reshape(E, 1)
    grid = (N // T,)
    logits_t, idx_t, wts_t = pl.pallas_call(
        _router_kernel,
        grid=grid,
        in_specs=[
            pl.BlockSpec((T, D), lambda i: (i, 0)),
            pl.BlockSpec((E, D), lambda i: (0, 0)),
            pl.BlockSpec((E, 1), lambda i: (0, 0)),
        ],
        out_specs=[
            pl.BlockSpec((E, T), lambda i: (0, i)),
            pl.BlockSpec((K, T), lambda i: (0, i)),
            pl.BlockSpec((K, T), lambda i: (0, i)),
        ],
        out_shape=[
            jax.ShapeDtypeStruct((E, N), jnp.float32),
            jax.ShapeDtypeStruct((K, N), jnp.int32),
            jax.ShapeDtypeStruct((K, N), jnp.float32),
        ],
    )(x_flat, weight.astype(jnp.float32), bias_col)
    return (idx_t.T, wts_t.T, logits_t.T)


# speedup vs baseline: 2.3779x; 1.0975x over previous
import functools

import jax
import jax.numpy as jnp
from jax.experimental import pallas as pl

K = 8


def _router_kernel(x_ref, w_ref, b_ref, logits_ref, idx_ref, wts_ref):
    T = x_ref.shape[0]
    E = w_ref.shape[0]
    dn = (((1,), (1,)), ((), ()))
    logits_t = jax.lax.dot_general(
        w_ref[...], x_ref[...], dn, preferred_element_type=jnp.float32)
    logits_ref[...] = logits_t
    s = jax.nn.sigmoid(logits_t)
    sel = s + b_ref[...]
    iota = jax.lax.broadcasted_iota(jnp.int32, (E, T), 0)
    work = sel
    rows_i = []
    rows_w = []
    for _ in range(K):
        m = jnp.max(work, axis=0, keepdims=True)
        am = jnp.min(jnp.where(work == m, iota, E), axis=0, keepdims=True)
        onehot = iota == am
        rows_i.append(am)
        rows_w.append(jnp.sum(jnp.where(onehot, s, 0.0), axis=0, keepdims=True))
        work = jnp.where(onehot, -jnp.inf, work)
    idx = jnp.concatenate(rows_i, axis=0)
    wts = jnp.concatenate(rows_w, axis=0)
    wts = wts / (jnp.sum(wts, axis=0, keepdims=True) + 1e-20)
    idx_ref[...] = idx
    wts_ref[...] = wts


@functools.partial(jax.jit, static_argnames=())
def kernel(x, weight, balance_bias):
    orig_shape = x.shape
    D = orig_shape[-1]
    x_flat = x.reshape(-1, D).astype(jnp.float32)
    N = x_flat.shape[0]
    E = weight.shape[0]
    T = 1024
    if N % T != 0:
        T = N
    bias_col = balance_bias.astype(jnp.float32).reshape(E, 1)
    grid = (N // T,)
    logits_t, idx_t, wts_t = pl.pallas_call(
        _router_kernel,
        grid=grid,
        in_specs=[
            pl.BlockSpec((T, D), lambda i: (i, 0)),
            pl.BlockSpec((E, D), lambda i: (0, 0)),
            pl.BlockSpec((E, 1), lambda i: (0, 0)),
        ],
        out_specs=[
            pl.BlockSpec((E, T), lambda i: (0, i)),
            pl.BlockSpec((K, T), lambda i: (0, i)),
            pl.BlockSpec((K, T), lambda i: (0, i)),
        ],
        out_shape=[
            jax.ShapeDtypeStruct((E, N), jnp.float32),
            jax.ShapeDtypeStruct((K, N), jnp.int32),
            jax.ShapeDtypeStruct((K, N), jnp.float32),
        ],
    )(x_flat, weight.astype(jnp.float32), bias_col)
    return (idx_t.T, wts_t.T, logits_t.T)
